# Initial kernel scaffold; baseline (speedup 1.0000x reference)
import jax
import jax.numpy as jnp
from jax.experimental import pallas as pl


def kernel(token_inp, pos_inp, turn_inp, token_table, pos_table, turn_table, text_embedding):
    raise NotImplementedError("write your pallas kernel here")


# speedup vs baseline: 1.7092x; 1.7092x over previous
import functools

import jax
import jax.numpy as jnp
from jax import lax
from jax.experimental import pallas as pl
from jax.experimental.pallas import tpu as pltpu
from jax.experimental.pallas import tpu_sc as plsc

HIDDEN = 64
NC = 2
NS = 16
NW = NC * NS
CHUNK = 128


@functools.lru_cache(maxsize=None)
def _build(N):
    n_w = N // NW
    n_chunks = n_w // CHUNK
    mesh = plsc.VectorSubcoreMesh(core_axis_name="c", subcore_axis_name="s")

    @functools.partial(
        pl.kernel,
        mesh=mesh,
        compiler_params=pltpu.CompilerParams(use_tc_tiling_on_sc=False),
        out_type=jax.ShapeDtypeStruct((N, HIDDEN), jnp.float32),
        scratch_types=[
            pltpu.VMEM((n_w,), jnp.int32),
            pltpu.VMEM((n_w,), jnp.int32),
            pltpu.VMEM((n_w,), jnp.int32),
            pltpu.VMEM((HIDDEN,), jnp.float32),
            pltpu.VMEM((CHUNK, HIDDEN), jnp.float32),
            pltpu.VMEM((CHUNK, HIDDEN), jnp.float32),
            pltpu.VMEM((CHUNK, HIDDEN), jnp.float32),
            pltpu.SemaphoreType.DMA,
        ],
    )
    def k(tok_i_hbm, pos_i_hbm, turn_i_hbm,
          tok_t_hbm, pos_t_hbm, turn_t_hbm, te_hbm,
          out_hbm,
          tok_idx, pos_idx, turn_idx, te_v, tok_v, pos_v, turn_v, sem):
        wid = lax.axis_index("s") * NC + lax.axis_index("c")
        base = wid * n_w
        pltpu.sync_copy(tok_i_hbm.at[pl.ds(base, n_w)], tok_idx)
        pltpu.sync_copy(pos_i_hbm.at[pl.ds(base, n_w)], pos_idx)
        pltpu.sync_copy(turn_i_hbm.at[pl.ds(base, n_w)], turn_idx)
        pltpu.sync_copy(te_hbm, te_v)

        def chunk_body(g, carry):
            off = g * CHUNK
            c1 = pltpu.async_copy(
                tok_t_hbm.at[tok_idx.at[pl.ds(off, CHUNK)]], tok_v, sem)
            c2 = pltpu.async_copy(
                pos_t_hbm.at[pos_idx.at[pl.ds(off, CHUNK)]], pos_v, sem)
            c3 = pltpu.async_copy(
                turn_t_hbm.at[turn_idx.at[pl.ds(off, CHUNK)]], turn_v, sem)
            c1.wait()
            c2.wait()
            c3.wait()

            def row_body(i, c):
                for j in range(HIDDEN // 16):
                    sl = pl.ds(j * 16, 16)
                    tok_v[i, sl] = (tok_v[i, sl] + pos_v[i, sl]
                                    + turn_v[i, sl] + te_v[sl])
                return c

            lax.fori_loop(0, CHUNK, row_body, 0)
            pltpu.sync_copy(tok_v, out_hbm.at[pl.ds(base + off, CHUNK)])
            return carry

        lax.fori_loop(0, n_chunks, chunk_body, 0)

    return k


def kernel(token_inp, pos_inp, turn_inp, token_table, pos_table, turn_table,
           text_embedding):
    B, L = token_inp.shape
    N = B * L
    out = _build(N)(
        token_inp.reshape(N), pos_inp.reshape(N), turn_inp.reshape(N),
        token_table, pos_table, turn_table, text_embedding)
    return out.reshape(B, L, HIDDEN)


# speedup vs baseline: 1.7227x; 1.0079x over previous
import functools

import jax
import jax.numpy as jnp
from jax import lax
from jax.experimental import pallas as pl
from jax.experimental.pallas import tpu as pltpu
from jax.experimental.pallas import tpu_sc as plsc

HIDDEN = 64
NC = 2
NS = 16
NW = NC * NS
CHUNK = 128


@functools.lru_cache(maxsize=None)
def _build(N):
    n_w = N // NW
    n_chunks = n_w // CHUNK
    n_pairs = n_chunks // 2
    mesh = plsc.VectorSubcoreMesh(core_axis_name="c", subcore_axis_name="s")

    row_buf = pltpu.VMEM((CHUNK, HIDDEN), jnp.float32)

    @functools.partial(
        pl.kernel,
        mesh=mesh,
        compiler_params=pltpu.CompilerParams(use_tc_tiling_on_sc=False),
        out_type=jax.ShapeDtypeStruct((N, HIDDEN), jnp.float32),
        scratch_types=[
            pltpu.VMEM((n_w,), jnp.int32),
            pltpu.VMEM((n_w,), jnp.int32),
            pltpu.VMEM((n_w,), jnp.int32),
            pltpu.VMEM((HIDDEN,), jnp.float32),
            [row_buf] * 4,
            [row_buf] * 4,
            [pltpu.SemaphoreType.DMA] * 4,
        ],
    )
    def k(tok_i_hbm, pos_i_hbm, turn_i_hbm,
          tok_t_hbm, pos_t_hbm, turn_t_hbm, te_hbm,
          out_hbm,
          tok_idx, pos_idx, turn_idx, te_v, slot_a, slot_b, sems):
        wid = lax.axis_index("s") * NC + lax.axis_index("c")
        base = wid * n_w
        pltpu.sync_copy(tok_i_hbm.at[pl.ds(base, n_w)], tok_idx)
        pltpu.sync_copy(pos_i_hbm.at[pl.ds(base, n_w)], pos_idx)
        pltpu.sync_copy(turn_i_hbm.at[pl.ds(base, n_w)], turn_idx)
        pltpu.sync_copy(te_hbm, te_v)
        g_sem = sems[:2]
        s_sem = sems[2:]
        slots = (slot_a, slot_b)

        def issue3(s, g):
            tokv, posv, turnv, _ = slots[s]
            off = g * CHUNK
            pltpu.async_copy(tok_t_hbm.at[tok_idx.at[pl.ds(off, CHUNK)]],
                             tokv, g_sem[s])
            pltpu.async_copy(pos_t_hbm.at[pos_idx.at[pl.ds(off, CHUNK)]],
                             posv, g_sem[s])
            pltpu.async_copy(turn_t_hbm.at[turn_idx.at[pl.ds(off, CHUNK)]],
                             turnv, g_sem[s])

        def drain_gathers(s):
            for buf in slots[s][:3]:
                pltpu.make_async_copy(out_hbm.at[pl.ds(0, CHUNK)],
                                      buf, g_sem[s]).wait()

        def drain_store(s):
            pltpu.make_async_copy(slots[s][3], out_hbm.at[pl.ds(0, CHUNK)],
                                  s_sem[s]).wait()

        def compute(s):
            tokv, posv, turnv, accv = slots[s]
            te = tuple(te_v[pl.ds(j * 16, 16)] for j in range(HIDDEN // 16))

            def row_body(i, te_c):
                for j in range(HIDDEN // 16):
                    sl = pl.ds(j * 16, 16)
                    accv[i, sl] = (tokv[i, sl] + posv[i, sl]
                                   + turnv[i, sl] + te_c[j])
                return te_c

            lax.fori_loop(0, CHUNK, row_body, te)

        def store(s, g):
            pltpu.async_copy(slots[s][3],
                             out_hbm.at[pl.ds(base + g * CHUNK, CHUNK)],
                             s_sem[s])

        issue3(0, 0)
        issue3(1, 1)

        def pair_body(kk, carry):
            for s in range(2):
                g = 2 * kk + s
                drain_gathers(s)

                @pl.when(kk > 0)
                def _():
                    drain_store(s)

                compute(s)

                @pl.when(kk < n_pairs - 1)
                def _():
                    issue3(s, g + 2)

                store(s, g)
            return carry

        lax.fori_loop(0, n_pairs, pair_body, 0)
        drain_store(0)
        drain_store(1)

    return k


def kernel(token_inp, pos_inp, turn_inp, token_table, pos_table, turn_table,
           text_embedding):
    B, L = token_inp.shape
    N = B * L
    out = _build(N)(
        token_inp.reshape(N), pos_inp.reshape(N), turn_inp.reshape(N),
        token_table, pos_table, turn_table, text_embedding)
    return out.reshape(B, L, HIDDEN)
